# baseline (speedup 1.0000x reference)
import functools

import jax
import jax.numpy as jnp
from jax import lax
from jax.experimental import pallas as pl
from jax.experimental.pallas import tpu as pltpu
from jax.experimental.pallas import tpu_sc as plsc

N, C, H, W = 4, 96, 384, 384
NB = 2
HW = H * W
NUM_WORKERS = 32
WPB = NUM_WORKERS // NB
SEG = HW // WPB
P = 96
NCHUNK = SEG // P
NC2 = NCHUNK // 2
LANES = 16
NG = P // LANES


def _floorf(x):
    big = jnp.abs(x) >= 8388608.0
    xs = jnp.where(big, 0.0, x)
    t = xs.astype(jnp.int32).astype(jnp.float32)
    return jnp.where(big, x, t - jnp.where(x < t, 1.0, 0.0))


def _sc_warp(src_t, flow):
    mesh = plsc.VectorSubcoreMesh(core_axis_name="c", subcore_axis_name="s")

    idx_t = pltpu.VMEM((P,), jnp.int32)
    w_t = pltpu.VMEM((P,), jnp.float32)
    rows_t = pltpu.VMEM((P, C), jnp.float32)
    ot_t = pltpu.VMEM((C, P + 1), jnp.float32)

    @functools.partial(
        pl.kernel,
        mesh=mesh,
        compiler_params=pltpu.CompilerParams(
            use_tc_tiling_on_sc=False, needs_layout_passes=False),
        out_type=jax.ShapeDtypeStruct((NB, C, HW), jnp.float32),
        scratch_types=[
            pltpu.VMEM((2, 2, P), jnp.float32),
            [[idx_t] * 4, [idx_t] * 4],
            [[w_t] * 4, [w_t] * 4],
            [[rows_t] * 4, [rows_t] * 4],
            [ot_t, ot_t],
            [pltpu.SemaphoreType.DMA] * 2,
            pltpu.SemaphoreType.DMA,
            [pltpu.SemaphoreType.DMA] * 2,
        ],
    )
    def warp(src_hbm, flow_hbm, out_hbm,
             flr, idx, wgt, rows, ot, semg, semf, semo):
        cid = lax.axis_index("c")
        sid = lax.axis_index("s")
        wid = sid * 2 + cid
        b = wid // WPB
        base0 = (wid % WPB) * SEG
        boff = b * HW
        iota = lax.iota(jnp.int32, LANES)

        def flow_src(t):
            return flow_hbm.at[b, :, pl.ds(base0 + t * P, P)]

        def compute_idx(t, par):
            i0, i1, i2, i3 = idx[par]
            w0, w1, w2, w3 = wgt[par]
            fl = flr.at[par]
            for g in range(NG):
                off = t * P + g * LANES
                g0 = base0 + off
                grow = g0 // W
                gcol = g0 - grow * W
                gy = jnp.full((LANES,), grow, jnp.int32)
                gx = gcol + iota
                gsl = pl.ds(g * LANES, LANES)
                Y = gy.astype(jnp.float32) + fl[0, gsl]
                X = gx.astype(jnp.float32) + fl[1, gsl]
                iyf = ((2.0 * (Y / 383.0 - 0.5)) + 1.0) / 2.0 * 383.0
                ixf = ((2.0 * (X / 383.0 - 0.5)) + 1.0) / 2.0 * 383.0
                fy0 = _floorf(iyf)
                fx0 = _floorf(ixf)
                x0 = jnp.clip(fx0, 0.0, W - 1.0).astype(jnp.int32)
                x1 = jnp.clip(fx0 + 1.0, 0.0, W - 1.0).astype(jnp.int32)
                y0 = jnp.clip(fy0, 0.0, H - 1.0).astype(jnp.int32)
                y1 = jnp.clip(fy0 + 1.0, 0.0, H - 1.0).astype(jnp.int32)
                row0 = boff + y0 * W
                row1 = boff + y1 * W
                i0[gsl] = row0 + x0
                i1[gsl] = row0 + x1
                i2[gsl] = row1 + x0
                i3[gsl] = row1 + x1
                wy1 = iyf - fy0
                wy0 = fy0 + 1.0 - iyf
                wx1 = ixf - fx0
                wx0 = fx0 + 1.0 - ixf
                w0[gsl] = wx0 * wy0
                w1[gsl] = wx1 * wy0
                w2[gsl] = wx0 * wy1
                w3[gsl] = wx1 * wy1

        def fire_gathers(par):
            for k in range(4):
                pltpu.async_copy(src_hbm.at[idx[par][k]], rows[par][k],
                                 semg[par])

        def wait_gathers(par):
            for k in range(4):
                pltpu.make_async_copy(src_hbm.at[idx[par][k]], rows[par][k],
                                      semg[par]).wait()

        def combine(par):
            r0_, r1_, r2_, r3_ = rows[par]
            w0, w1, w2, w3 = wgt[par]
            o = ot[par]
            chvs = [cg * LANES + iota for cg in range(C // LANES)]

            @plsc.parallel_loop(0, P, 1, unroll=4)
            def px(j):
                jv = jnp.full((LANES,), j, jnp.int32)
                s0 = plsc.load_gather(w0, [jv])
                s1 = plsc.load_gather(w1, [jv])
                s2 = plsc.load_gather(w2, [jv])
                s3 = plsc.load_gather(w3, [jv])
                for cg in range(C // LANES):
                    csl = pl.ds(cg * LANES, LANES)
                    v = (r0_[j, csl] * s0 + r1_[j, csl] * s1
                         + r2_[j, csl] * s2 + r3_[j, csl] * s3)
                    plsc.store_scatter(o, [chvs[cg], jv], v)

        def out_dst(t):
            return out_hbm.at[b, :, pl.ds(base0 + t * P, P)]

        pltpu.async_copy(flow_src(0), flr.at[0], semf)
        pltpu.make_async_copy(flow_src(0), flr.at[0], semf).wait()
        pltpu.async_copy(flow_src(1), flr.at[1], semf)
        compute_idx(0, 0)
        fire_gathers(0)

        def step(t2, carry):
            for p in range(2):
                t = 2 * t2 + p
                a = p
                bb = 1 - p
                more = (t2 < NC2 - 1) if p == 1 else None

                def wait_flow():
                    pltpu.make_async_copy(flow_src(t + 1), flr.at[bb],
                                          semf).wait()

                def fire_flow():
                    pltpu.async_copy(flow_src(t + 2), flr.at[a], semf)

                def prep_next():
                    compute_idx(t + 1, bb)
                    fire_gathers(bb)

                if p == 0:
                    wait_flow()
                    lax.cond(t2 < NC2 - 1, fire_flow, lambda: None)
                    prep_next()
                else:
                    def do_all():
                        wait_flow()
                        lax.cond(t2 < NC2 - 1, fire_flow, lambda: None)
                        prep_next()
                    lax.cond(more, do_all, lambda: None)

                wait_gathers(a)

                def wait_out():
                    pltpu.make_async_copy(ot[a].at[:, pl.ds(0, P)],
                                          out_dst(t), semo[a]).wait()
                lax.cond(t2 > 0, wait_out, lambda: None)

                combine(a)
                pltpu.async_copy(ot[a].at[:, pl.ds(0, P)], out_dst(t),
                                 semo[a])
            return carry

        lax.fori_loop(0, NC2, step, 0)

        pltpu.make_async_copy(ot[0].at[:, pl.ds(0, P)],
                              out_dst(NCHUNK - 2), semo[0]).wait()
        pltpu.make_async_copy(ot[1].at[:, pl.ds(0, P)],
                              out_dst(NCHUNK - 1), semo[1]).wait()

    return warp(src_t, flow)


def kernel(src, flow):
    halves = []
    for i in range(N // NB):
        sl = slice(i * NB, (i + 1) * NB)
        tbl = (src[sl].reshape(NB, C, HW).transpose(0, 2, 1)
               .reshape(NB * HW, C))
        o = _sc_warp(tbl, flow[sl].reshape(NB, 2, HW))
        halves.append(o.reshape(NB, C, H, W))
    return jnp.concatenate(halves, axis=0)


# speedup vs baseline: 1.1451x; 1.1451x over previous
import functools

import jax
import jax.numpy as jnp
from jax import lax
from jax.experimental import pallas as pl
from jax.experimental.pallas import tpu as pltpu
from jax.experimental.pallas import tpu_sc as plsc

N, C, H, W = 4, 96, 384, 384
NB = 4
HW = H * W
NUM_WORKERS = 32
WPB = NUM_WORKERS // NB
SEG = HW // WPB
P = 96
NCHUNK = SEG // P
NC2 = NCHUNK // 2
LANES = 16
NG = P // LANES


def _floorf(x):
    big = jnp.abs(x) >= 8388608.0
    xs = jnp.where(big, 0.0, x)
    t = xs.astype(jnp.int32).astype(jnp.float32)
    return jnp.where(big, x, t - jnp.where(x < t, 1.0, 0.0))


def _sc_warp(src_t, flow):
    mesh = plsc.VectorSubcoreMesh(core_axis_name="c", subcore_axis_name="s")

    idx_t = pltpu.VMEM((P,), jnp.int32)
    w_t = pltpu.VMEM((P,), jnp.float32)
    rows_t = pltpu.VMEM((P, C), jnp.float32)
    ot_t = pltpu.VMEM((C, P + 1), jnp.float32)

    @functools.partial(
        pl.kernel,
        mesh=mesh,
        compiler_params=pltpu.CompilerParams(
            use_tc_tiling_on_sc=False, needs_layout_passes=False),
        out_type=jax.ShapeDtypeStruct((NB, C, HW), jnp.float32),
        scratch_types=[
            pltpu.VMEM((2, 2, P), jnp.float32),
            [[idx_t] * 4, [idx_t] * 4],
            [[w_t] * 4, [w_t] * 4],
            [[rows_t] * 4, [rows_t] * 4],
            [ot_t, ot_t],
            [pltpu.SemaphoreType.DMA] * 2,
            pltpu.SemaphoreType.DMA,
            [pltpu.SemaphoreType.DMA] * 2,
        ],
    )
    def warp(src_hbm, flow_hbm, out_hbm,
             flr, idx, wgt, rows, ot, semg, semf, semo):
        cid = lax.axis_index("c")
        sid = lax.axis_index("s")
        wid = sid * 2 + cid
        b = wid // WPB
        base0 = (wid % WPB) * SEG
        boff = b * HW
        iota = lax.iota(jnp.int32, LANES)

        def flow_src(t):
            return flow_hbm.at[b, :, pl.ds(base0 + t * P, P)]

        def compute_idx(t, par):
            i0, i1, i2, i3 = idx[par]
            w0, w1, w2, w3 = wgt[par]
            fl = flr.at[par]
            for g in range(NG):
                off = t * P + g * LANES
                g0 = base0 + off
                grow = g0 // W
                gcol = g0 - grow * W
                gy = jnp.full((LANES,), grow, jnp.int32)
                gx = gcol + iota
                gsl = pl.ds(g * LANES, LANES)
                Y = gy.astype(jnp.float32) + fl[0, gsl]
                X = gx.astype(jnp.float32) + fl[1, gsl]
                iyf = ((2.0 * (Y / 383.0 - 0.5)) + 1.0) / 2.0 * 383.0
                ixf = ((2.0 * (X / 383.0 - 0.5)) + 1.0) / 2.0 * 383.0
                fy0 = _floorf(iyf)
                fx0 = _floorf(ixf)
                x0 = jnp.clip(fx0, 0.0, W - 1.0).astype(jnp.int32)
                x1 = jnp.clip(fx0 + 1.0, 0.0, W - 1.0).astype(jnp.int32)
                y0 = jnp.clip(fy0, 0.0, H - 1.0).astype(jnp.int32)
                y1 = jnp.clip(fy0 + 1.0, 0.0, H - 1.0).astype(jnp.int32)
                row0 = boff + y0 * W
                row1 = boff + y1 * W
                i0[gsl] = row0 + x0
                i1[gsl] = row0 + x1
                i2[gsl] = row1 + x0
                i3[gsl] = row1 + x1
                wy1 = iyf - fy0
                wy0 = fy0 + 1.0 - iyf
                wx1 = ixf - fx0
                wx0 = fx0 + 1.0 - ixf
                w0[gsl] = wx0 * wy0
                w1[gsl] = wx1 * wy0
                w2[gsl] = wx0 * wy1
                w3[gsl] = wx1 * wy1

        def fire_gathers(par):
            for k in range(4):
                pltpu.async_copy(src_hbm.at[idx[par][k]], rows[par][k],
                                 semg[par])

        def wait_gathers(par):
            for k in range(4):
                pltpu.make_async_copy(src_hbm.at[idx[par][k]], rows[par][k],
                                      semg[par]).wait()

        def combine(par):
            r0_, r1_, r2_, r3_ = rows[par]
            w0, w1, w2, w3 = wgt[par]
            o = ot[par]
            chvs = [cg * LANES + iota for cg in range(C // LANES)]

            @plsc.parallel_loop(0, P, 1, unroll=4)
            def px(j):
                jv = jnp.full((LANES,), j, jnp.int32)
                s0 = plsc.load_gather(w0, [jv])
                s1 = plsc.load_gather(w1, [jv])
                s2 = plsc.load_gather(w2, [jv])
                s3 = plsc.load_gather(w3, [jv])
                for cg in range(C // LANES):
                    csl = pl.ds(cg * LANES, LANES)
                    v = (r0_[j, csl] * s0 + r1_[j, csl] * s1
                         + r2_[j, csl] * s2 + r3_[j, csl] * s3)
                    plsc.store_scatter(o, [chvs[cg], jv], v)

        def out_dst(t):
            return out_hbm.at[b, :, pl.ds(base0 + t * P, P)]

        pltpu.async_copy(flow_src(0), flr.at[0], semf)
        pltpu.make_async_copy(flow_src(0), flr.at[0], semf).wait()
        pltpu.async_copy(flow_src(1), flr.at[1], semf)
        compute_idx(0, 0)
        fire_gathers(0)

        def step(t2, carry):
            for p in range(2):
                t = 2 * t2 + p
                a = p
                bb = 1 - p
                more = (t2 < NC2 - 1) if p == 1 else None

                def wait_flow():
                    pltpu.make_async_copy(flow_src(t + 1), flr.at[bb],
                                          semf).wait()

                def fire_flow():
                    pltpu.async_copy(flow_src(t + 2), flr.at[a], semf)

                def prep_next():
                    compute_idx(t + 1, bb)
                    fire_gathers(bb)

                if p == 0:
                    wait_flow()
                    lax.cond(t2 < NC2 - 1, fire_flow, lambda: None)
                    prep_next()
                else:
                    def do_all():
                        wait_flow()
                        lax.cond(t2 < NC2 - 1, fire_flow, lambda: None)
                        prep_next()
                    lax.cond(more, do_all, lambda: None)

                wait_gathers(a)

                def wait_out():
                    pltpu.make_async_copy(ot[a].at[:, pl.ds(0, P)],
                                          out_dst(t), semo[a]).wait()
                lax.cond(t2 > 0, wait_out, lambda: None)

                combine(a)
                pltpu.async_copy(ot[a].at[:, pl.ds(0, P)], out_dst(t),
                                 semo[a])
            return carry

        lax.fori_loop(0, NC2, step, 0)

        pltpu.make_async_copy(ot[0].at[:, pl.ds(0, P)],
                              out_dst(NCHUNK - 2), semo[0]).wait()
        pltpu.make_async_copy(ot[1].at[:, pl.ds(0, P)],
                              out_dst(NCHUNK - 1), semo[1]).wait()

    return warp(src_t, flow)


def kernel(src, flow):
    halves = []
    for i in range(N // NB):
        sl = slice(i * NB, (i + 1) * NB)
        tbl = (src[sl].reshape(NB, C, HW).transpose(0, 2, 1)
               .reshape(NB * HW, C))
        o = _sc_warp(tbl, flow[sl].reshape(NB, 2, HW))
        halves.append(o.reshape(NB, C, H, W))
    return jnp.concatenate(halves, axis=0)


# speedup vs baseline: 1.1684x; 1.0204x over previous
import functools

import jax
import jax.numpy as jnp
from jax import lax
from jax.experimental import pallas as pl
from jax.experimental.pallas import tpu as pltpu
from jax.experimental.pallas import tpu_sc as plsc

N, C, H, W = 4, 96, 384, 384
NB = 4
HW = H * W
NUM_WORKERS = 32
WPB = NUM_WORKERS // NB
SEG = HW // WPB
P = 128
NCHUNK = SEG // P
NC2 = NCHUNK // 2
LANES = 16
NG = P // LANES


def _floorf(x):
    big = jnp.abs(x) >= 8388608.0
    xs = jnp.where(big, 0.0, x)
    t = xs.astype(jnp.int32).astype(jnp.float32)
    return jnp.where(big, x, t - jnp.where(x < t, 1.0, 0.0))


def _sc_warp(src_t, flow):
    mesh = plsc.VectorSubcoreMesh(core_axis_name="c", subcore_axis_name="s")

    idx_t = pltpu.VMEM((P,), jnp.int32)
    w_t = pltpu.VMEM((P,), jnp.float32)
    rows_t = pltpu.VMEM((P, C), jnp.float32)
    ot_t = pltpu.VMEM((C, P + 1), jnp.float32)

    @functools.partial(
        pl.kernel,
        mesh=mesh,
        compiler_params=pltpu.CompilerParams(
            use_tc_tiling_on_sc=False, needs_layout_passes=False),
        out_type=jax.ShapeDtypeStruct((NB, C, HW), jnp.float32),
        scratch_types=[
            pltpu.VMEM((2, 2, P), jnp.float32),
            [[idx_t] * 4, [idx_t] * 4],
            [[w_t] * 4, [w_t] * 4],
            [[rows_t] * 4, [rows_t] * 4],
            [ot_t, ot_t],
            [pltpu.SemaphoreType.DMA] * 2,
            pltpu.SemaphoreType.DMA,
            [pltpu.SemaphoreType.DMA] * 2,
        ],
    )
    def warp(src_hbm, flow_hbm, out_hbm,
             flr, idx, wgt, rows, ot, semg, semf, semo):
        cid = lax.axis_index("c")
        sid = lax.axis_index("s")
        wid = sid * 2 + cid
        b = wid // WPB
        base0 = (wid % WPB) * SEG
        boff = b * HW
        iota = lax.iota(jnp.int32, LANES)

        def flow_src(t):
            return flow_hbm.at[b, :, pl.ds(base0 + t * P, P)]

        def compute_idx(t, par):
            i0, i1, i2, i3 = idx[par]
            w0, w1, w2, w3 = wgt[par]
            fl = flr.at[par]
            for g in range(NG):
                off = t * P + g * LANES
                g0 = base0 + off
                grow = g0 // W
                gcol = g0 - grow * W
                gy = jnp.full((LANES,), grow, jnp.int32)
                gx = gcol + iota
                gsl = pl.ds(g * LANES, LANES)
                Y = gy.astype(jnp.float32) + fl[0, gsl]
                X = gx.astype(jnp.float32) + fl[1, gsl]
                iyf = ((2.0 * (Y / 383.0 - 0.5)) + 1.0) / 2.0 * 383.0
                ixf = ((2.0 * (X / 383.0 - 0.5)) + 1.0) / 2.0 * 383.0
                fy0 = _floorf(iyf)
                fx0 = _floorf(ixf)
                x0 = jnp.clip(fx0, 0.0, W - 1.0).astype(jnp.int32)
                x1 = jnp.clip(fx0 + 1.0, 0.0, W - 1.0).astype(jnp.int32)
                y0 = jnp.clip(fy0, 0.0, H - 1.0).astype(jnp.int32)
                y1 = jnp.clip(fy0 + 1.0, 0.0, H - 1.0).astype(jnp.int32)
                row0 = boff + y0 * W
                row1 = boff + y1 * W
                i0[gsl] = row0 + x0
                i1[gsl] = row0 + x1
                i2[gsl] = row1 + x0
                i3[gsl] = row1 + x1
                wy1 = iyf - fy0
                wy0 = fy0 + 1.0 - iyf
                wx1 = ixf - fx0
                wx0 = fx0 + 1.0 - ixf
                w0[gsl] = wx0 * wy0
                w1[gsl] = wx1 * wy0
                w2[gsl] = wx0 * wy1
                w3[gsl] = wx1 * wy1

        def fire_gathers(par):
            for k in range(4):
                pltpu.async_copy(src_hbm.at[idx[par][k]], rows[par][k],
                                 semg[par])

        def wait_gathers(par):
            for k in range(4):
                pltpu.make_async_copy(src_hbm.at[idx[par][k]], rows[par][k],
                                      semg[par]).wait()

        def combine(par):
            r0_, r1_, r2_, r3_ = rows[par]
            w0, w1, w2, w3 = wgt[par]
            o = ot[par]
            chvs = [cg * LANES + iota for cg in range(C // LANES)]

            @plsc.parallel_loop(0, P, 1, unroll=4)
            def px(j):
                jv = jnp.full((LANES,), j, jnp.int32)
                s0 = plsc.load_gather(w0, [jv])
                s1 = plsc.load_gather(w1, [jv])
                s2 = plsc.load_gather(w2, [jv])
                s3 = plsc.load_gather(w3, [jv])
                for cg in range(C // LANES):
                    csl = pl.ds(cg * LANES, LANES)
                    v = (r0_[j, csl] * s0 + r1_[j, csl] * s1
                         + r2_[j, csl] * s2 + r3_[j, csl] * s3)
                    plsc.store_scatter(o, [chvs[cg], jv], v)

        def out_dst(t):
            return out_hbm.at[b, :, pl.ds(base0 + t * P, P)]

        pltpu.async_copy(flow_src(0), flr.at[0], semf)
        pltpu.make_async_copy(flow_src(0), flr.at[0], semf).wait()
        pltpu.async_copy(flow_src(1), flr.at[1], semf)
        compute_idx(0, 0)
        fire_gathers(0)

        def step(t2, carry):
            for p in range(2):
                t = 2 * t2 + p
                a = p
                bb = 1 - p
                more = (t2 < NC2 - 1) if p == 1 else None

                def wait_flow():
                    pltpu.make_async_copy(flow_src(t + 1), flr.at[bb],
                                          semf).wait()

                def fire_flow():
                    pltpu.async_copy(flow_src(t + 2), flr.at[a], semf)

                def prep_next():
                    compute_idx(t + 1, bb)
                    fire_gathers(bb)

                if p == 0:
                    wait_flow()
                    lax.cond(t2 < NC2 - 1, fire_flow, lambda: None)
                    prep_next()
                else:
                    def do_all():
                        wait_flow()
                        lax.cond(t2 < NC2 - 1, fire_flow, lambda: None)
                        prep_next()
                    lax.cond(more, do_all, lambda: None)

                wait_gathers(a)

                def wait_out():
                    pltpu.make_async_copy(ot[a].at[:, pl.ds(0, P)],
                                          out_dst(t), semo[a]).wait()
                lax.cond(t2 > 0, wait_out, lambda: None)

                combine(a)
                pltpu.async_copy(ot[a].at[:, pl.ds(0, P)], out_dst(t),
                                 semo[a])
            return carry

        lax.fori_loop(0, NC2, step, 0)

        pltpu.make_async_copy(ot[0].at[:, pl.ds(0, P)],
                              out_dst(NCHUNK - 2), semo[0]).wait()
        pltpu.make_async_copy(ot[1].at[:, pl.ds(0, P)],
                              out_dst(NCHUNK - 1), semo[1]).wait()

    return warp(src_t, flow)


def kernel(src, flow):
    halves = []
    for i in range(N // NB):
        sl = slice(i * NB, (i + 1) * NB)
        tbl = (src[sl].reshape(NB, C, HW).transpose(0, 2, 1)
               .reshape(NB * HW, C))
        o = _sc_warp(tbl, flow[sl].reshape(NB, 2, HW))
        halves.append(o.reshape(NB, C, H, W))
    return jnp.concatenate(halves, axis=0)
